# Initial kernel scaffold; baseline (speedup 1.0000x reference)
import jax
import jax.numpy as jnp
from jax.experimental import pallas as pl


def kernel(tokens, W_E):
    raise NotImplementedError("write your pallas kernel here")


# speedup vs baseline: 1.5687x; 1.5687x over previous
import functools

import jax
import jax.numpy as jnp
from jax import lax
from jax.experimental import pallas as pl
from jax.experimental.pallas import tpu as pltpu
from jax.experimental.pallas import tpu_sc as plsc

D_MODEL = 1024
NUM_CORES = 2
NUM_SUBCORES = 16
NUM_WORKERS = NUM_CORES * NUM_SUBCORES
CHUNK = 64


def _make_emb_kernel(n_tokens: int):
    tokens_per_worker = n_tokens // NUM_WORKERS
    n_chunks = tokens_per_worker // CHUNK

    mesh = plsc.VectorSubcoreMesh(
        core_axis_name="c", subcore_axis_name="s"
    )

    @functools.partial(
        pl.kernel,
        mesh=mesh,
        out_type=jax.ShapeDtypeStruct((n_tokens, D_MODEL), jnp.float32),
        scratch_types=[
            pltpu.VMEM((n_chunks, CHUNK), jnp.int32),
            pltpu.VMEM((CHUNK, D_MODEL), jnp.float32),
            pltpu.SemaphoreType.DMA,
        ],
    )
    def emb(tokens_hbm, table_hbm, out_hbm, idx_v, rows_v, sem):
        wid = lax.axis_index("s") * NUM_CORES + lax.axis_index("c")
        base = wid * tokens_per_worker
        pltpu.sync_copy(tokens_hbm.at[wid], idx_v)

        def body(j, carry):
            pltpu.async_copy(table_hbm.at[idx_v.at[j]], rows_v, sem).wait()
            pltpu.sync_copy(rows_v, out_hbm.at[pl.ds(base + j * CHUNK, CHUNK)])
            return carry

        lax.fori_loop(0, n_chunks, body, 0, unroll=False)

    return emb


def kernel(tokens, W_E):
    batch, seq_len = tokens.shape
    n_tokens = batch * seq_len
    tokens_per_worker = n_tokens // NUM_WORKERS
    n_chunks = tokens_per_worker // CHUNK
    tok3 = tokens.reshape(NUM_WORKERS, n_chunks, CHUNK).astype(jnp.int32)
    out = _make_emb_kernel(n_tokens)(tok3, W_E)
    return out.reshape(batch, seq_len, W_E.shape[1])
